# Initial kernel scaffold; baseline (speedup 1.0000x reference)
import jax
import jax.numpy as jnp
from jax.experimental import pallas as pl


def kernel(x, e, global_attr, node_graph_ids, edge_graph_ids, W, b):
    raise NotImplementedError("write your pallas kernel here")


# speedup vs baseline: 6.4606x; 6.4606x over previous
import functools

import jax
import jax.numpy as jnp
from jax.experimental import pallas as pl
from jax.experimental.pallas import tpu as pltpu

NUM_GRAPHS = 64
E_ROWS = 320000
N_ROWS = 10000
E_FEATS = 16
X_FEATS = 128
OUT_FEATS = 128

GRID = 50
E_BLK = E_ROWS // GRID
N_GRID = 25
N_BLK = N_ROWS // N_GRID


def _body(eids_ref, nids_ref, e_ref, x_ref, g_ref, w_ref, b_ref, out_ref,
          acc_e, acc_x):
    step = pl.program_id(0)

    @pl.when(step == 0)
    def _init():
        acc_e[...] = jnp.zeros_like(acc_e)
        acc_x[...] = jnp.zeros_like(acc_x)

    gids = jax.lax.broadcasted_iota(jnp.int32, (NUM_GRAPHS, E_BLK), 0)
    onehot_e = (eids_ref[0, 0, :][None, :] == gids).astype(jnp.float32)
    acc_e[...] += jax.lax.dot(onehot_e, e_ref[...],
                              preferred_element_type=jnp.float32)

    @pl.when(step < N_GRID)
    def _node():
        ngids = jax.lax.broadcasted_iota(jnp.int32, (NUM_GRAPHS, N_BLK), 0)
        onehot_n = (nids_ref[0, 0, :][None, :] == ngids).astype(jnp.float32)
        acc_x[...] += jax.lax.dot(onehot_n, x_ref[...],
                                  preferred_element_type=jnp.float32)

    @pl.when(step == GRID - 1)
    def _linear():
        w = w_ref[...]
        out = jax.lax.dot(acc_e[...], w[0:E_FEATS, :],
                          preferred_element_type=jnp.float32)
        out += jax.lax.dot(acc_x[...], w[E_FEATS:E_FEATS + X_FEATS, :],
                           preferred_element_type=jnp.float32)
        out += jax.lax.dot(g_ref[...], w[E_FEATS + X_FEATS:, :],
                           preferred_element_type=jnp.float32)
        out_ref[...] = out + b_ref[0, :][None, :]


@jax.jit
def _global_block(x, e, global_attr, node_ids, edge_ids, W, b):
    eids3 = edge_ids.astype(jnp.int32).reshape(GRID, 1, E_BLK)
    nids3 = node_ids.astype(jnp.int32).reshape(N_GRID, 1, N_BLK)
    b2 = b.reshape(1, OUT_FEATS)

    return pl.pallas_call(
        _body,
        grid=(GRID,),
        in_specs=[
            pl.BlockSpec((1, 1, E_BLK), lambda i: (i, 0, 0)),
            pl.BlockSpec((1, 1, N_BLK),
                         lambda i: (jnp.minimum(i, N_GRID - 1), 0, 0)),
            pl.BlockSpec((E_BLK, E_FEATS), lambda i: (i, 0)),
            pl.BlockSpec((N_BLK, X_FEATS),
                         lambda i: (jnp.minimum(i, N_GRID - 1), 0)),
            pl.BlockSpec((NUM_GRAPHS, X_FEATS), lambda i: (0, 0)),
            pl.BlockSpec((E_FEATS + X_FEATS + X_FEATS, OUT_FEATS),
                         lambda i: (0, 0)),
            pl.BlockSpec((1, OUT_FEATS), lambda i: (0, 0)),
        ],
        out_specs=pl.BlockSpec((NUM_GRAPHS, OUT_FEATS), lambda i: (0, 0)),
        out_shape=jax.ShapeDtypeStruct((NUM_GRAPHS, OUT_FEATS), jnp.float32),
        scratch_shapes=[
            pltpu.VMEM((NUM_GRAPHS, E_FEATS), jnp.float32),
            pltpu.VMEM((NUM_GRAPHS, X_FEATS), jnp.float32),
        ],
        compiler_params=pltpu.CompilerParams(
            dimension_semantics=("arbitrary",),
        ),
    )(eids3, nids3, e, x, global_attr, W, b2)


def kernel(x, e, global_attr, node_graph_ids, edge_graph_ids, W, b):
    return _global_block(x, e, global_attr, node_graph_ids, edge_graph_ids,
                         W, b)
